# baseline (speedup 1.0000x reference)
import jax
import jax.numpy as jnp
from jax import lax
from jax.experimental import pallas as pl
from jax.experimental.pallas import tpu as pltpu
from jax.experimental.pallas import tpu_sc as plsc

VOCAB = 1000
VPAD = 1024
N_TOK = 1024 * 50

NC = 2
NS = 16
L = 16
NW = NC * NS

TBLK = 128
NTB = N_TOK // TBLK
KMAX = -(-NTB // NW)
CCOL = 40
NCC = VOCAB // CCOL
CGRP = TBLK // L


def _sc_body(ttab_hbm, idx_hbm, tgt_hbm, lse_hbm,
             logitsT_hbm, psum_hbm,
             idx_v, tgt_v, stage_v, reg0_v, reg1_v, lse_v, acc_v,
             rsem0, rsem1):
    wid = lax.axis_index("s") * NC + lax.axis_index("c")
    regs = (reg0_v, reg1_v)
    rsems = (rsem0, rsem1)

    blk = []
    for k in range(KMAX):
        b = jnp.minimum(wid + k * NW, NTB - 1)
        blk.append(b)
        pltpu.sync_copy(idx_hbm.at[b], idx_v.at[k])
        pltpu.sync_copy(tgt_hbm.at[b], tgt_v.at[k])
    pltpu.sync_copy(lse_hbm, lse_v)

    acc = jnp.zeros((L,), jnp.float32)
    for k in range(KMAX):
        valid = (wid + k * NW) < NTB
        vmask = jnp.full((L,), valid)
        for g in range(CGRP):
            ids = idx_v[k, pl.ds(g * L, L)]
            lse16 = plsc.load_gather(lse_v, [ids])
            acc = acc + jnp.where(vmask, lse16, 0.0)

    def region(cc, k, wait_first):
        reg_v = regs[k % 2]
        sem = rsems[k % 2]
        b = blk[k]
        dst = logitsT_hbm.at[pl.ds(cc * CCOL, CCOL), pl.ds(b * TBLK, TBLK)]
        if wait_first:
            pltpu.make_async_copy(reg_v, dst, sem).wait()
        idx16 = [idx_v[k, pl.ds(g * L, L)] for g in range(CGRP)]

        @plsc.parallel_loop(0, CCOL, unroll=8)
        def row_body(r):
            for g in range(CGRP):
                vals = plsc.load_gather(stage_v, [idx16[g] + r * VPAD])
                reg_v[r, pl.ds(g * L, L)] = vals

        pltpu.async_copy(reg_v, dst, sem)

    def nll_tgt(cc, k, acc):
        valid = (wid + k * NW) < NTB
        for g in range(CGRP):
            ids = idx_v[k, pl.ds(g * L, L)]
            tgs = tgt_v[k, pl.ds(g * L, L)]
            loc = tgs - cc * CCOL
            m = jnp.logical_and(loc >= 0, loc < CCOL)
            m = jnp.logical_and(m, jnp.full((L,), valid))
            loc_s = jnp.where(m, loc, 0)
            tv = plsc.load_gather(stage_v, [loc_s * VPAD + ids])
            acc = acc - jnp.where(m, tv, 0.0)
        return acc

    def chunk(cc, acc, first):
        pltpu.sync_copy(ttab_hbm.at[pl.ds(cc * CCOL * VPAD, CCOL * VPAD)],
                        stage_v)
        for k in range(KMAX):
            region(cc, k, wait_first=(not first) or k >= 2)
            acc = nll_tgt(cc, k, acc)
        return acc

    acc = chunk(0, acc, first=True)
    acc = lax.fori_loop(1, NCC, lambda cc, a: chunk(cc, a, False), acc)

    for k in (KMAX - 2, KMAX - 1):
        reg_v = regs[k % 2]
        dst = logitsT_hbm.at[pl.ds((NCC - 1) * CCOL, CCOL),
                             pl.ds(blk[k] * TBLK, TBLK)]
        pltpu.make_async_copy(reg_v, dst, rsems[k % 2]).wait()

    acc_v[0, :] = acc
    for z in range(1, 8):
        acc_v[z, :] = jnp.zeros((L,), jnp.float32)
    pltpu.sync_copy(acc_v, psum_hbm.at[wid])


_sc_call = pl.kernel(
    _sc_body,
    mesh=plsc.VectorSubcoreMesh(core_axis_name="c", subcore_axis_name="s"),
    compiler_params=pltpu.CompilerParams(
        use_tc_tiling_on_sc=True, needs_layout_passes=False),
    out_type=[
        jax.ShapeDtypeStruct((VOCAB, N_TOK), jnp.float32),
        jax.ShapeDtypeStruct((NW, 8, L), jnp.float32),
    ],
    name="sc_gather_ce",
    scratch_types=[
        pltpu.VMEM((KMAX, TBLK), jnp.int32),
        pltpu.VMEM((KMAX, TBLK), jnp.int32),
        pltpu.VMEM((CCOL * VPAD,), jnp.float32),
        pltpu.VMEM((CCOL, TBLK), jnp.float32),
        pltpu.VMEM((CCOL, TBLK), jnp.float32),
        pltpu.VMEM((VPAD,), jnp.float32),
        pltpu.VMEM((8, L), jnp.float32),
        pltpu.SemaphoreType.DMA,
        pltpu.SemaphoreType.DMA,
    ],
)


def _lse_body(table_ref, lse_ref):
    t = table_ref[...]
    m = jnp.max(t, axis=1, keepdims=True)
    s = jnp.sum(jnp.exp(t - m), axis=1, keepdims=True)
    lse_ref[...] = m + jnp.log(s)


_lse_call = pl.pallas_call(
    _lse_body,
    out_shape=jax.ShapeDtypeStruct((VOCAB, 1), jnp.float32),
)


def kernel(idx, targets, table):
    idx_f = idx.reshape(NTB, TBLK).astype(jnp.int32)
    tgt_f = targets.reshape(NTB, TBLK).astype(jnp.int32)
    table = table.astype(jnp.float32)
    ttab = jnp.pad(table.T, ((0, 0), (0, VPAD - VOCAB))).reshape(-1)
    lse = jnp.pad(_lse_call(table).reshape(VOCAB), (0, VPAD - VOCAB))
    logitsT, psum = _sc_call(ttab, idx_f, tgt_f, lse)
    loss = jnp.sum(psum) / jnp.float32(N_TOK)
    return (logitsT.T, loss)


# speedup vs baseline: 1.2001x; 1.2001x over previous
import jax
import jax.numpy as jnp
from jax import lax
from jax.experimental import pallas as pl
from jax.experimental.pallas import tpu as pltpu
from jax.experimental.pallas import tpu_sc as plsc

VOCAB = 1000
VPAD = 1024
N_TOK = 1024 * 50

NC = 2
NS = 16
L = 16
NW = NC * NS

TBLK = 128
NTB = N_TOK // TBLK
KMAX = -(-NTB // NW)
CCOL = 40
NCC = VOCAB // CCOL
CGRP = TBLK // L


def _sc_body(ttab_hbm, idx_hbm, tgt_hbm, lse_hbm,
             logitsT_hbm, psum_hbm,
             idx_v, tgt_v, stage0_v, stage1_v, reg0_v, reg1_v, lse_v, acc_v,
             rsem0, rsem1, ssem0, ssem1):
    wid = lax.axis_index("s") * NC + lax.axis_index("c")
    regs = (reg0_v, reg1_v)
    rsems = (rsem0, rsem1)
    stages = (stage0_v, stage1_v)
    ssems = (ssem0, ssem1)

    blk = []
    for k in range(KMAX):
        b = jnp.minimum(wid + k * NW, NTB - 1)
        blk.append(b)
        pltpu.sync_copy(idx_hbm.at[b], idx_v.at[k])
        pltpu.sync_copy(tgt_hbm.at[b], tgt_v.at[k])
    pltpu.sync_copy(lse_hbm, lse_v)

    acc = jnp.zeros((L,), jnp.float32)
    for k in range(KMAX):
        valid = (wid + k * NW) < NTB
        vmask = jnp.full((L,), valid)
        for g in range(CGRP):
            ids = idx_v[k, pl.ds(g * L, L)]
            lse16 = plsc.load_gather(lse_v, [ids])
            acc = acc + jnp.where(vmask, lse16, 0.0)

    def region(cc, k, stage_v, wait_first):
        reg_v = regs[k % 2]
        sem = rsems[k % 2]
        b = blk[k]
        dst = logitsT_hbm.at[pl.ds(cc * CCOL, CCOL), pl.ds(b * TBLK, TBLK)]
        if wait_first:
            pltpu.make_async_copy(reg_v, dst, sem).wait()
        idx16 = [idx_v[k, pl.ds(g * L, L)] for g in range(CGRP)]

        @plsc.parallel_loop(0, CCOL, unroll=4)
        def row_body(r):
            for g in range(CGRP):
                vals = plsc.load_gather(stage_v, [idx16[g] + r * VPAD])
                reg_v[r, pl.ds(g * L, L)] = vals

        pltpu.async_copy(reg_v, dst, sem)

    def nll_tgt(cc, k, stage_v, acc):
        valid = (wid + k * NW) < NTB
        for g in range(CGRP):
            ids = idx_v[k, pl.ds(g * L, L)]
            tgs = tgt_v[k, pl.ds(g * L, L)]
            loc = tgs - cc * CCOL
            m = jnp.logical_and(loc >= 0, loc < CCOL)
            m = jnp.logical_and(m, jnp.full((L,), valid))
            loc_s = jnp.where(m, loc, 0)
            tv = plsc.load_gather(stage_v, [loc_s * VPAD + ids])
            acc = acc - jnp.where(m, tv, 0.0)
        return acc

    def stage_start(cc, sb):
        return pltpu.async_copy(
            ttab_hbm.at[pl.ds(cc * CCOL * VPAD, CCOL * VPAD)],
            stages[sb], ssems[sb])

    def chunk(cc, sb, acc, first):
        pltpu.make_async_copy(
            ttab_hbm.at[pl.ds(cc * CCOL * VPAD, CCOL * VPAD)],
            stages[sb], ssems[sb]).wait()

        @pl.when(cc + 1 < NCC)
        def _():
            stage_start(cc + 1, 1 - sb)

        for k in range(KMAX):
            region(cc, k, stages[sb], wait_first=(not first) or k >= 2)
            acc = nll_tgt(cc, k, stages[sb], acc)
        return acc

    stage_start(0, 0)
    acc = chunk(0, 0, acc, first=True)

    def pair_body(pi, a):
        cc = 1 + 2 * pi
        a = chunk(cc, 1, a, False)
        return chunk(cc + 1, 0, a, False)

    acc = lax.fori_loop(0, (NCC - 1) // 2, pair_body, acc)

    for k in (KMAX - 2, KMAX - 1):
        reg_v = regs[k % 2]
        dst = logitsT_hbm.at[pl.ds((NCC - 1) * CCOL, CCOL),
                             pl.ds(blk[k] * TBLK, TBLK)]
        pltpu.make_async_copy(reg_v, dst, rsems[k % 2]).wait()

    acc_v[0, :] = acc
    for z in range(1, 8):
        acc_v[z, :] = jnp.zeros((L,), jnp.float32)
    pltpu.sync_copy(acc_v, psum_hbm.at[wid])


_sc_call = pl.kernel(
    _sc_body,
    mesh=plsc.VectorSubcoreMesh(core_axis_name="c", subcore_axis_name="s"),
    compiler_params=pltpu.CompilerParams(
        use_tc_tiling_on_sc=True, needs_layout_passes=False),
    out_type=[
        jax.ShapeDtypeStruct((VOCAB, N_TOK), jnp.float32),
        jax.ShapeDtypeStruct((NW, 8, L), jnp.float32),
    ],
    name="sc_gather_ce",
    scratch_types=[
        pltpu.VMEM((KMAX, TBLK), jnp.int32),
        pltpu.VMEM((KMAX, TBLK), jnp.int32),
        pltpu.VMEM((CCOL * VPAD,), jnp.float32),
        pltpu.VMEM((CCOL * VPAD,), jnp.float32),
        pltpu.VMEM((CCOL, TBLK), jnp.float32),
        pltpu.VMEM((CCOL, TBLK), jnp.float32),
        pltpu.VMEM((VPAD,), jnp.float32),
        pltpu.VMEM((8, L), jnp.float32),
        pltpu.SemaphoreType.DMA,
        pltpu.SemaphoreType.DMA,
        pltpu.SemaphoreType.DMA,
        pltpu.SemaphoreType.DMA,
    ],
)


def _lse_body(table_ref, lse_ref):
    t = table_ref[...]
    m = jnp.max(t, axis=1, keepdims=True)
    s = jnp.sum(jnp.exp(t - m), axis=1, keepdims=True)
    lse_ref[...] = m + jnp.log(s)


_lse_call = pl.pallas_call(
    _lse_body,
    out_shape=jax.ShapeDtypeStruct((VOCAB, 1), jnp.float32),
)


def kernel(idx, targets, table):
    idx_f = idx.reshape(NTB, TBLK).astype(jnp.int32)
    tgt_f = targets.reshape(NTB, TBLK).astype(jnp.int32)
    table = table.astype(jnp.float32)
    ttab = jnp.pad(table.T, ((0, 0), (0, VPAD - VOCAB))).reshape(-1)
    lse = jnp.pad(_lse_call(table).reshape(VOCAB), (0, VPAD - VOCAB))
    logitsT, psum = _sc_call(ttab, idx_f, tgt_f, lse)
    loss = jnp.sum(psum) / jnp.float32(N_TOK)
    return (logitsT.T, loss)


# speedup vs baseline: 1.2060x; 1.0049x over previous
import jax
import jax.numpy as jnp
from jax import lax
from jax.experimental import pallas as pl
from jax.experimental.pallas import tpu as pltpu
from jax.experimental.pallas import tpu_sc as plsc

VOCAB = 1000
VPAD = 1024
N_TOK = 1024 * 50

NC = 2
NS = 16
L = 16
NW = NC * NS

TBLK = 128
NTB = N_TOK // TBLK
KMAX = -(-NTB // NW)
CCOL = 40
NCC = VOCAB // CCOL
CGRP = TBLK // L


def _sc_body(ttab_hbm, idx_hbm, tgt_hbm, lse_hbm,
             logitsT_hbm, psum_hbm,
             idx_v, tgt_v, stage0_v, stage1_v,
             reg0_v, reg1_v, reg2_v, reg3_v, lse_v, acc_v,
             rsem0, rsem1, rsem2, rsem3, ssem0, ssem1):
    wid = lax.axis_index("s") * NC + lax.axis_index("c")
    regs = (reg0_v, reg1_v, reg2_v, reg3_v)
    rsems = (rsem0, rsem1, rsem2, rsem3)
    stages = (stage0_v, stage1_v)
    ssems = (ssem0, ssem1)

    blk = []
    for k in range(KMAX):
        b = jnp.minimum(wid + k * NW, NTB - 1)
        blk.append(b)
        pltpu.sync_copy(idx_hbm.at[b], idx_v.at[k])
        pltpu.sync_copy(tgt_hbm.at[b], tgt_v.at[k])
    pltpu.sync_copy(lse_hbm, lse_v)

    acc = jnp.zeros((L,), jnp.float32)
    for k in range(KMAX):
        valid = (wid + k * NW) < NTB
        vmask = jnp.full((L,), valid)
        for g in range(CGRP):
            ids = idx_v[k, pl.ds(g * L, L)]
            lse16 = plsc.load_gather(lse_v, [ids])
            acc = acc + jnp.where(vmask, lse16, 0.0)

    def region(cc, k, stage_v, wait_first):
        reg_v = regs[k % 4]
        sem = rsems[k % 4]
        b = blk[k]
        dst = logitsT_hbm.at[pl.ds(cc * CCOL, CCOL), pl.ds(b * TBLK, TBLK)]
        if wait_first:
            pltpu.make_async_copy(reg_v, dst, sem).wait()
        idx16 = [idx_v[k, pl.ds(g * L, L)] for g in range(CGRP)]

        @plsc.parallel_loop(0, CCOL, unroll=4)
        def row_body(r):
            for g in range(CGRP):
                vals = plsc.load_gather(stage_v, [idx16[g] + r * VPAD])
                reg_v[r, pl.ds(g * L, L)] = vals

        pltpu.async_copy(reg_v, dst, sem)

    def nll_tgt(cc, k, stage_v, acc):
        valid = (wid + k * NW) < NTB
        for g in range(CGRP):
            ids = idx_v[k, pl.ds(g * L, L)]
            tgs = tgt_v[k, pl.ds(g * L, L)]
            loc = tgs - cc * CCOL
            m = jnp.logical_and(loc >= 0, loc < CCOL)
            m = jnp.logical_and(m, jnp.full((L,), valid))
            loc_s = jnp.where(m, loc, 0)
            tv = plsc.load_gather(stage_v, [loc_s * VPAD + ids])
            acc = acc - jnp.where(m, tv, 0.0)
        return acc

    def stage_start(cc, sb):
        return pltpu.async_copy(
            ttab_hbm.at[pl.ds(cc * CCOL * VPAD, CCOL * VPAD)],
            stages[sb], ssems[sb])

    def chunk(cc, sb, acc, first):
        pltpu.make_async_copy(
            ttab_hbm.at[pl.ds(cc * CCOL * VPAD, CCOL * VPAD)],
            stages[sb], ssems[sb]).wait()

        @pl.when(cc + 1 < NCC)
        def _():
            stage_start(cc + 1, 1 - sb)

        for k in range(KMAX):
            region(cc, k, stages[sb], wait_first=(not first) or k >= 4)
            acc = nll_tgt(cc, k, stages[sb], acc)
        return acc

    stage_start(0, 0)
    acc = chunk(0, 0, acc, first=True)

    def pair_body(pi, a):
        cc = 1 + 2 * pi
        a = chunk(cc, 1, a, False)
        return chunk(cc + 1, 0, a, False)

    acc = lax.fori_loop(0, (NCC - 1) // 2, pair_body, acc)

    for k in range(KMAX - 4, KMAX):
        reg_v = regs[k % 4]
        dst = logitsT_hbm.at[pl.ds((NCC - 1) * CCOL, CCOL),
                             pl.ds(blk[k] * TBLK, TBLK)]
        pltpu.make_async_copy(reg_v, dst, rsems[k % 4]).wait()

    acc_v[0, :] = acc
    for z in range(1, 8):
        acc_v[z, :] = jnp.zeros((L,), jnp.float32)
    pltpu.sync_copy(acc_v, psum_hbm.at[wid])


_sc_call = pl.kernel(
    _sc_body,
    mesh=plsc.VectorSubcoreMesh(core_axis_name="c", subcore_axis_name="s"),
    compiler_params=pltpu.CompilerParams(
        use_tc_tiling_on_sc=True, needs_layout_passes=False),
    out_type=[
        jax.ShapeDtypeStruct((VOCAB, N_TOK), jnp.float32),
        jax.ShapeDtypeStruct((NW, 8, L), jnp.float32),
    ],
    name="sc_gather_ce",
    scratch_types=[
        pltpu.VMEM((KMAX, TBLK), jnp.int32),
        pltpu.VMEM((KMAX, TBLK), jnp.int32),
        pltpu.VMEM((CCOL * VPAD,), jnp.float32),
        pltpu.VMEM((CCOL * VPAD,), jnp.float32),
        pltpu.VMEM((CCOL, TBLK), jnp.float32),
        pltpu.VMEM((CCOL, TBLK), jnp.float32),
        pltpu.VMEM((CCOL, TBLK), jnp.float32),
        pltpu.VMEM((CCOL, TBLK), jnp.float32),
        pltpu.VMEM((VPAD,), jnp.float32),
        pltpu.VMEM((8, L), jnp.float32),
        pltpu.SemaphoreType.DMA,
        pltpu.SemaphoreType.DMA,
        pltpu.SemaphoreType.DMA,
        pltpu.SemaphoreType.DMA,
        pltpu.SemaphoreType.DMA,
        pltpu.SemaphoreType.DMA,
    ],
)


def _lse_body(table_ref, lse_ref):
    t = table_ref[...]
    m = jnp.max(t, axis=1, keepdims=True)
    s = jnp.sum(jnp.exp(t - m), axis=1, keepdims=True)
    lse_ref[...] = m + jnp.log(s)


_lse_call = pl.pallas_call(
    _lse_body,
    out_shape=jax.ShapeDtypeStruct((VOCAB, 1), jnp.float32),
)


def kernel(idx, targets, table):
    idx_f = idx.reshape(NTB, TBLK).astype(jnp.int32)
    tgt_f = targets.reshape(NTB, TBLK).astype(jnp.int32)
    table = table.astype(jnp.float32)
    ttab = jnp.pad(table.T, ((0, 0), (0, VPAD - VOCAB))).reshape(-1)
    lse = jnp.pad(_lse_call(table).reshape(VOCAB), (0, VPAD - VOCAB))
    logitsT, psum = _sc_call(ttab, idx_f, tgt_f, lse)
    loss = jnp.sum(psum) / jnp.float32(N_TOK)
    return (logitsT.T, loss)


# speedup vs baseline: 1.2587x; 1.0437x over previous
import jax
import jax.numpy as jnp
from jax import lax
from jax.experimental import pallas as pl
from jax.experimental.pallas import tpu as pltpu
from jax.experimental.pallas import tpu_sc as plsc

VOCAB = 1000
VPAD = 1024
N_TOK = 1024 * 50

NC = 2
NS = 16
L = 16
NW = NC * NS

TBLK = 128
NTB = N_TOK // TBLK
KMAX = -(-NTB // NW)
CCOL = 40
NCC = VOCAB // CCOL
CGRP = TBLK // L


def _sc_body(ttab_hbm, idx_hbm, tgt_hbm, lse_hbm,
             logitsT_hbm, psum_hbm,
             idx_v, tgt_v, stage0_v, stage1_v,
             reg0_v, reg1_v, reg2_v, reg3_v, lse_v, acc_v,
             rsem0, rsem1, rsem2, rsem3, ssem0, ssem1):
    wid = lax.axis_index("s") * NC + lax.axis_index("c")
    regs = (reg0_v, reg1_v, reg2_v, reg3_v)
    rsems = (rsem0, rsem1, rsem2, rsem3)
    stages = (stage0_v, stage1_v)
    ssems = (ssem0, ssem1)

    blk = []
    for k in range(KMAX):
        b = jnp.minimum(wid + k * NW, NTB - 1)
        blk.append(b)
        pltpu.async_copy(idx_hbm.at[b], idx_v.at[k], ssem0)
        pltpu.async_copy(tgt_hbm.at[b], tgt_v.at[k], ssem0)
    pltpu.async_copy(lse_hbm, lse_v, ssem0)
    for k in range(KMAX):
        pltpu.make_async_copy(idx_hbm.at[blk[k]], idx_v.at[k], ssem0).wait()
        pltpu.make_async_copy(tgt_hbm.at[blk[k]], tgt_v.at[k], ssem0).wait()
    pltpu.make_async_copy(lse_hbm, lse_v, ssem0).wait()

    acc = jnp.zeros((L,), jnp.float32)
    for k in range(KMAX):
        valid = (wid + k * NW) < NTB
        vmask = jnp.full((L,), valid)
        for g in range(CGRP):
            ids = idx_v[k, pl.ds(g * L, L)]
            lse16 = plsc.load_gather(lse_v, [ids])
            acc = acc + jnp.where(vmask, lse16, 0.0)

    def region(cc, k, stage_v, wait_first):
        reg_v = regs[k % 4]
        sem = rsems[k % 4]
        b = blk[k]
        dst = logitsT_hbm.at[pl.ds(cc * CCOL, CCOL), pl.ds(b * TBLK, TBLK)]
        if wait_first:
            pltpu.make_async_copy(reg_v, dst, sem).wait()
        idx16 = [idx_v[k, pl.ds(g * L, L)] for g in range(CGRP)]

        @plsc.parallel_loop(0, CCOL, unroll=4)
        def row_body(r):
            for g in range(CGRP):
                vals = plsc.load_gather(stage_v, [idx16[g] + r * VPAD])
                reg_v[r, pl.ds(g * L, L)] = vals

        pltpu.async_copy(reg_v, dst, sem)

    def nll_tgt(cc, k, stage_v, acc):
        valid = (wid + k * NW) < NTB
        for g in range(CGRP):
            ids = idx_v[k, pl.ds(g * L, L)]
            tgs = tgt_v[k, pl.ds(g * L, L)]
            loc = tgs - cc * CCOL
            m = jnp.logical_and(loc >= 0, loc < CCOL)
            m = jnp.logical_and(m, jnp.full((L,), valid))
            loc_s = jnp.where(m, loc, 0)
            tv = plsc.load_gather(stage_v, [loc_s * VPAD + ids])
            acc = acc - jnp.where(m, tv, 0.0)
        return acc

    def stage_start(cc, sb):
        return pltpu.async_copy(
            ttab_hbm.at[pl.ds(cc * CCOL * VPAD, CCOL * VPAD)],
            stages[sb], ssems[sb])

    def chunk(cc, sb, acc, first):
        pltpu.make_async_copy(
            ttab_hbm.at[pl.ds(cc * CCOL * VPAD, CCOL * VPAD)],
            stages[sb], ssems[sb]).wait()

        @pl.when(cc + 1 < NCC)
        def _():
            stage_start(cc + 1, 1 - sb)

        for k in range(KMAX):
            region(cc, k, stages[sb], wait_first=(not first) or k >= 4)
            acc = nll_tgt(cc, k, stages[sb], acc)
        return acc

    stage_start(0, 0)
    acc = chunk(0, 0, acc, first=True)

    def pair_body(pi, a):
        cc = 1 + 2 * pi
        a = chunk(cc, 1, a, False)
        return chunk(cc + 1, 0, a, False)

    acc = lax.fori_loop(0, (NCC - 1) // 2, pair_body, acc)

    for k in range(KMAX - 4, KMAX):
        reg_v = regs[k % 4]
        dst = logitsT_hbm.at[pl.ds((NCC - 1) * CCOL, CCOL),
                             pl.ds(blk[k] * TBLK, TBLK)]
        pltpu.make_async_copy(reg_v, dst, rsems[k % 4]).wait()

    acc_v[0, :] = acc
    for z in range(1, 8):
        acc_v[z, :] = jnp.zeros((L,), jnp.float32)
    pltpu.sync_copy(acc_v, psum_hbm.at[wid])


_sc_call = pl.kernel(
    _sc_body,
    mesh=plsc.VectorSubcoreMesh(core_axis_name="c", subcore_axis_name="s"),
    compiler_params=pltpu.CompilerParams(
        use_tc_tiling_on_sc=True, needs_layout_passes=False),
    out_type=[
        jax.ShapeDtypeStruct((VOCAB, N_TOK), jnp.float32),
        jax.ShapeDtypeStruct((NW, 8, L), jnp.float32),
    ],
    name="sc_gather_ce",
    scratch_types=[
        pltpu.VMEM((KMAX, TBLK), jnp.int32),
        pltpu.VMEM((KMAX, TBLK), jnp.int32),
        pltpu.VMEM((CCOL * VPAD,), jnp.float32),
        pltpu.VMEM((CCOL * VPAD,), jnp.float32),
        pltpu.VMEM((CCOL, TBLK), jnp.float32),
        pltpu.VMEM((CCOL, TBLK), jnp.float32),
        pltpu.VMEM((CCOL, TBLK), jnp.float32),
        pltpu.VMEM((CCOL, TBLK), jnp.float32),
        pltpu.VMEM((VPAD,), jnp.float32),
        pltpu.VMEM((8, L), jnp.float32),
        pltpu.SemaphoreType.DMA,
        pltpu.SemaphoreType.DMA,
        pltpu.SemaphoreType.DMA,
        pltpu.SemaphoreType.DMA,
        pltpu.SemaphoreType.DMA,
        pltpu.SemaphoreType.DMA,
    ],
)


def _lse_body(table_ref, lse_ref):
    t = table_ref[...]
    m = jnp.max(t, axis=1, keepdims=True)
    s = jnp.sum(jnp.exp(t - m), axis=1, keepdims=True)
    lse_ref[...] = m + jnp.log(s)


_lse_call = pl.pallas_call(
    _lse_body,
    out_shape=jax.ShapeDtypeStruct((VOCAB, 1), jnp.float32),
)


def kernel(idx, targets, table):
    idx_f = idx.reshape(NTB, TBLK).astype(jnp.int32)
    tgt_f = targets.reshape(NTB, TBLK).astype(jnp.int32)
    table = table.astype(jnp.float32)
    ttab = jnp.pad(table.T, ((0, 0), (0, VPAD - VOCAB))).reshape(-1)
    lse = jnp.pad(_lse_call(table).reshape(VOCAB), (0, VPAD - VOCAB))
    logitsT, psum = _sc_call(ttab, idx_f, tgt_f, lse)
    loss = jnp.sum(psum) / jnp.float32(N_TOK)
    return (logitsT.T, loss)
